# baseline (speedup 1.0000x reference)
import functools

import jax
import jax.numpy as jnp
from jax import lax
from jax.experimental import pallas as pl
from jax.experimental.pallas import tpu as pltpu
from jax.experimental.pallas import tpu_sc as plsc

_NI = 2


def _greedy_sc(x):
    B, V, U = x.shape
    info = plsc.get_sparse_core_info()
    NC, NS, L = info.num_cores, info.num_subcores, info.num_lanes
    NW = NC * NS
    IPW = B // NW
    NI = _NI
    NREG = (U + L - 1) // L
    mesh = plsc.VectorSubcoreMesh(core_axis_name="c", subcore_axis_name="s")

    @functools.partial(
        pl.kernel,
        out_type=(
            jax.ShapeDtypeStruct((B,), jnp.float32),
            jax.ShapeDtypeStruct((B, V), jnp.int32),
        ),
        mesh=mesh,
        scratch_types=(
            [pltpu.VMEM((V, U), jnp.float32) for _ in range(NI)]
            + [pltpu.VMEM((V,), jnp.int32) for _ in range(NI)]
            + [pltpu.VMEM((IPW,), jnp.float32)]
        ),
        compiler_params=pltpu.CompilerParams(needs_layout_passes=False),
    )
    def greedy(x_hbm, size_hbm, seq_hbm, *bufs):
        wbufs = bufs[:NI]
        seqrows = bufs[NI : 2 * NI]
        sizebuf = bufs[2 * NI]
        wid = lax.axis_index("s") * NC + lax.axis_index("c")
        base = wid * IPW
        iota = lax.iota(jnp.int32, L)
        lane0 = iota == 0
        offs = [j * L for j in range(NREG - 1)] + [U - L]
        idxs = [iota + o for o in offs]

        @pl.loop(0, IPW // NI)
        def inst_loop(p):
            b = base + p * NI
            for k in range(NI):
                pltpu.sync_copy(x_hbm.at[b + k], wbufs[k])

            def step(t, carry):
                sizes, ms = carry
                t_vec = jnp.full((L,), t, dtype=jnp.int32)
                new_sizes = []
                new_ms = []
                for k in range(NI):
                    ws = [
                        wbufs[k][t, pl.ds(o, L)] + m
                        for o, m in zip(offs, ms[k])
                    ]
                    mx = ws[0]
                    for wv in ws[1:]:
                        mx = jnp.maximum(mx, wv)
                    gmax = lax.reduce_max(mx, (0,))
                    gmax_vec = jnp.full((L,), gmax, dtype=jnp.float32)
                    cand = [
                        jnp.where(wv == gmax_vec, iv, jnp.int32(4 * L * NREG))
                        for wv, iv in zip(ws, idxs)
                    ]
                    mn = cand[0]
                    for cv in cand[1:]:
                        mn = jnp.minimum(mn, cv)
                    sel = lax.reduce_min(mn, (0,))
                    sel_vec = jnp.full((L,), sel, dtype=jnp.int32)
                    plsc.store_scatter(
                        seqrows[k], [t_vec], sel_vec, mask=lane0
                    )
                    nz = sel_vec != 0
                    new_sizes.append(sizes[k] - jnp.where(nz, gmax_vec, 0.0))
                    sel_upd = jnp.where(nz, sel_vec, jnp.int32(-1))
                    new_ms.append(tuple(
                        jnp.where(iv == sel_upd, jnp.float32(-2.0), m)
                        for iv, m in zip(idxs, ms[k])
                    ))
                return tuple(new_sizes), tuple(new_ms)

            zero_v = jnp.zeros((L,), jnp.float32)
            carry = (
                tuple(zero_v for _ in range(NI)),
                tuple(tuple(zero_v for _ in range(NREG)) for _ in range(NI)),
            )
            sizes, _ = lax.fori_loop(0, V, step, carry)
            for k in range(NI):
                pltpu.sync_copy(seqrows[k], seq_hbm.at[b + k])
                plsc.store_scatter(
                    sizebuf,
                    [jnp.full((L,), p * NI + k, dtype=jnp.int32)],
                    sizes[k],
                    mask=lane0,
                )

        pltpu.sync_copy(sizebuf, size_hbm.at[pl.ds(base, IPW)])

    return greedy(x)


def kernel(x, u_size, v_size):
    del u_size, v_size
    neg_size, seqs = _greedy_sc(x)
    return neg_size, seqs


# speedup vs baseline: 1.3124x; 1.3124x over previous
import functools

import jax
import jax.numpy as jnp
from jax import lax
from jax.experimental import pallas as pl
from jax.experimental.pallas import tpu as pltpu
from jax.experimental.pallas import tpu_sc as plsc

_NI = 2


def _greedy_sc(x):
    B, V, U = x.shape
    info = plsc.get_sparse_core_info()
    NC, NS, L = info.num_cores, info.num_subcores, info.num_lanes
    NW = NC * NS
    IPW = B // NW
    NI = _NI
    NREG = (U + L - 1) // L
    VP = ((V + L - 1) // L) * L
    mesh = plsc.VectorSubcoreMesh(core_axis_name="c", subcore_axis_name="s")

    @functools.partial(
        pl.kernel,
        out_type=(
            jax.ShapeDtypeStruct((B,), jnp.float32),
            jax.ShapeDtypeStruct((B, V), jnp.int32),
        ),
        mesh=mesh,
        scratch_types=(
            [pltpu.VMEM((V, U), jnp.float32) for _ in range(NI)]
            + [pltpu.VMEM((V,), jnp.int32) for _ in range(NI)]
            + [pltpu.VMEM((IPW,), jnp.float32)]
        ),
        compiler_params=pltpu.CompilerParams(needs_layout_passes=False),
    )
    def greedy(x_hbm, size_hbm, seq_hbm, *bufs):
        wbufs = bufs[:NI]
        seqrows = bufs[NI : 2 * NI]
        sizebuf = bufs[2 * NI]
        wid = lax.axis_index("s") * NC + lax.axis_index("c")
        base = wid * IPW
        iota = lax.iota(jnp.int32, L)
        lane0 = iota == 0
        offs = [j * L for j in range(NREG - 1)] + [U - L]
        idxs_f = [(iota + o).astype(jnp.float32) for o in offs]
        big_f = jnp.float32(4 * L * NREG)

        @pl.loop(0, IPW // NI)
        def inst_loop(p):
            b = base + p * NI
            for k in range(NI):
                pltpu.sync_copy(x_hbm.at[b + k], wbufs[k])

            def step(t, carry):
                sizes, pends, ms = carry
                lane_t = jnp.full((L,), t % L, dtype=jnp.int32)
                is_lane_t = iota == lane_t
                new_sizes = []
                new_pends = []
                new_ms = []
                for k in range(NI):
                    ws = [
                        wbufs[k][t, pl.ds(o, L)] + m
                        for o, m in zip(offs, ms[k])
                    ]
                    mx = ws[0]
                    for wv in ws[1:]:
                        mx = jnp.maximum(mx, wv)
                    gmax = lax.reduce_max(mx, (0,))
                    gmax_vec = jnp.full((L,), gmax, dtype=jnp.float32)
                    cand = [
                        jnp.where(wv == gmax_vec, iv, big_f)
                        for wv, iv in zip(ws, idxs_f)
                    ]
                    mn = cand[0]
                    for cv in cand[1:]:
                        mn = jnp.minimum(mn, cv)
                    sel = lax.reduce_min(mn, (0,))
                    sel_vec = jnp.full((L,), sel, dtype=jnp.float32)
                    nz = sel_vec != 0.0
                    new_sizes.append(sizes[k] - jnp.where(nz, gmax_vec, 0.0))
                    new_pends.append(
                        jnp.where(is_lane_t, sel_vec, pends[k])
                    )
                    sel_upd = jnp.where(nz, sel_vec, jnp.float32(-1.0))
                    new_ms.append(tuple(
                        jnp.where(iv == sel_upd, jnp.float32(-2.0), m)
                        for iv, m in zip(idxs_f, ms[k])
                    ))

                @pl.when(t % L == L - 1)
                def _flush():
                    for k in range(NI):
                        seqrows[k][pl.ds(t - (L - 1), L)] = (
                            new_pends[k].astype(jnp.int32)
                        )

                return tuple(new_sizes), tuple(new_pends), tuple(new_ms)

            zero_v = jnp.zeros((L,), jnp.float32)
            carry = (
                tuple(zero_v for _ in range(NI)),
                tuple(zero_v for _ in range(NI)),
                tuple(tuple(zero_v for _ in range(NREG)) for _ in range(NI)),
            )
            sizes, pends, _ = lax.fori_loop(0, V, step, carry)
            for k in range(NI):
                if V % L:
                    plsc.store_scatter(
                        seqrows[k],
                        [iota + (V - V % L)],
                        pends[k].astype(jnp.int32),
                        mask=iota < (V % L),
                    )
                pltpu.sync_copy(seqrows[k], seq_hbm.at[b + k])
                plsc.store_scatter(
                    sizebuf,
                    [jnp.full((L,), p * NI + k, dtype=jnp.int32)],
                    sizes[k],
                    mask=lane0,
                )

        pltpu.sync_copy(sizebuf, size_hbm.at[pl.ds(base, IPW)])

    return greedy(x)


def kernel(x, u_size, v_size):
    del u_size, v_size
    neg_size, seqs = _greedy_sc(x)
    return neg_size, seqs


# speedup vs baseline: 1.6953x; 1.2917x over previous
import functools

import jax
import jax.numpy as jnp
from jax import lax
from jax.experimental import pallas as pl
from jax.experimental.pallas import tpu as pltpu
from jax.experimental.pallas import tpu_sc as plsc

_NI = 2


def _greedy_sc(x):
    B, V, U = x.shape
    info = plsc.get_sparse_core_info()
    NC, NS, L = info.num_cores, info.num_subcores, info.num_lanes
    NW = NC * NS
    IPW = B // NW
    NI = _NI
    NP = IPW // NI
    NREG = (U + L - 1) // L
    mesh = plsc.VectorSubcoreMesh(core_axis_name="c", subcore_axis_name="s")

    @functools.partial(
        pl.kernel,
        out_type=(
            jax.ShapeDtypeStruct((B,), jnp.float32),
            jax.ShapeDtypeStruct((B, V), jnp.int32),
        ),
        mesh=mesh,
        scratch_types=(
            [pltpu.VMEM((V, U), jnp.float32) for _ in range(2 * NI)]
            + [pltpu.VMEM((IPW, V), jnp.int32), pltpu.VMEM((IPW,), jnp.float32)]
            + [pltpu.SemaphoreType.DMA for _ in range(2 * NI)]
        ),
        compiler_params=pltpu.CompilerParams(needs_layout_passes=False),
    )
    def greedy(x_hbm, size_hbm, seq_hbm, *bufs):
        wbufs = (bufs[0:NI], bufs[NI : 2 * NI])
        seqbuf = bufs[2 * NI]
        sizebuf = bufs[2 * NI + 1]
        insems = (bufs[2 * NI + 2 : 3 * NI + 2], bufs[3 * NI + 2 : 4 * NI + 2])
        wid = lax.axis_index("s") * NC + lax.axis_index("c")
        base = wid * IPW
        iota = lax.iota(jnp.int32, L)
        lane0 = iota == 0
        offs = [j * L for j in range(NREG - 1)] + [U - L]
        idxs_f = [(iota + o).astype(jnp.float32) for o in offs]
        big_f = jnp.float32(4 * L * NREG)

        def start_in(slot, pair):
            bb = base + pair * NI
            for k in range(NI):
                pltpu.async_copy(
                    x_hbm.at[bb + k], wbufs[slot][k], insems[slot][k]
                )

        def wait_in(slot):
            for k in range(NI):
                pltpu.make_async_copy(
                    x_hbm.at[0], wbufs[slot][k], insems[slot][k]
                ).wait()

        def compute(slot, pair):
            def step(t, carry):
                sizes, pends, ms = carry
                lane_t = jnp.full((L,), t % L, dtype=jnp.int32)
                is_lane_t = iota == lane_t
                new_sizes = []
                new_pends = []
                new_ms = []
                for k in range(NI):
                    ws = [
                        wbufs[slot][k][t, pl.ds(o, L)] + m
                        for o, m in zip(offs, ms[k])
                    ]
                    mx = ws[0]
                    for wv in ws[1:]:
                        mx = jnp.maximum(mx, wv)
                    gmax = lax.reduce_max(mx, (0,))
                    gmax_vec = jnp.full((L,), gmax, dtype=jnp.float32)
                    cand = [
                        jnp.where(wv == gmax_vec, iv, big_f)
                        for wv, iv in zip(ws, idxs_f)
                    ]
                    mn = cand[0]
                    for cv in cand[1:]:
                        mn = jnp.minimum(mn, cv)
                    sel = lax.reduce_min(mn, (0,))
                    sel_vec = jnp.full((L,), sel, dtype=jnp.float32)
                    nz = sel_vec != 0.0
                    new_sizes.append(sizes[k] - gmax_vec)
                    new_pends.append(jnp.where(is_lane_t, sel_vec, pends[k]))
                    sel_upd = jnp.where(nz, sel_vec, jnp.float32(-1.0))
                    new_ms.append(tuple(
                        jnp.where(iv == sel_upd, jnp.float32(-2.0), m)
                        for iv, m in zip(idxs_f, ms[k])
                    ))

                @pl.when(t % L == L - 1)
                def _flush():
                    for k in range(NI):
                        seqbuf[pair * NI + k, pl.ds(t - (L - 1), L)] = (
                            new_pends[k].astype(jnp.int32)
                        )

                return tuple(new_sizes), tuple(new_pends), tuple(new_ms)

            zero_v = jnp.zeros((L,), jnp.float32)
            carry = (
                tuple(zero_v for _ in range(NI)),
                tuple(zero_v for _ in range(NI)),
                tuple(tuple(zero_v for _ in range(NREG)) for _ in range(NI)),
            )
            sizes, pends, _ = lax.fori_loop(0, V, step, carry)
            for k in range(NI):
                if V % L:
                    plsc.store_scatter(
                        seqbuf,
                        [
                            jnp.full((L,), pair * NI + k, dtype=jnp.int32),
                            iota + (V - V % L),
                        ],
                        pends[k].astype(jnp.int32),
                        mask=iota < (V % L),
                    )
                plsc.store_scatter(
                    sizebuf,
                    [jnp.full((L,), pair * NI + k, dtype=jnp.int32)],
                    sizes[k],
                    mask=lane0,
                )

        start_in(0, 0)

        @pl.loop(0, NP, step=2)
        def outer(p):
            wait_in(0)
            start_in(1, p + 1)
            compute(0, p)
            wait_in(1)

            @pl.when(p + 2 < NP)
            def _prefetch():
                start_in(0, p + 2)

            compute(1, p + 1)

        pltpu.sync_copy(seqbuf, seq_hbm.at[pl.ds(base, IPW)])
        pltpu.sync_copy(sizebuf, size_hbm.at[pl.ds(base, IPW)])

    return greedy(x)


def kernel(x, u_size, v_size):
    del u_size, v_size
    neg_size, seqs = _greedy_sc(x)
    return neg_size, seqs
